# speedup vs baseline: 3.7687x; 3.7687x over previous
import jax
import jax.numpy as jnp
from jax import lax
from jax.experimental import pallas as pl
from jax.experimental.pallas import tpu as pltpu

M = N = 4096
BS = 16
NB = M // BS
K = int(int(M * N * 0.05) / (BS * BS))
GRID = 16
ROWS = M // GRID


def _sel_matrix(rows, cols, fn):
    i = lax.broadcasted_iota(jnp.int32, (rows, cols), 0)
    j = lax.broadcasted_iota(jnp.int32, (rows, cols), 1)
    return jnp.where(fn(i, j), 1.0, 0.0).astype(jnp.float32)


def _dot(a, b):
    return lax.dot_general(a, b, (((1,), (0,)), ((), ())),
                           precision=lax.Precision.HIGHEST,
                           preferred_element_type=jnp.float32)


def _scores_mask_kernel(grad_ref, mask_ref, scores):
    i = pl.program_id(0)
    x = grad_ref[...]
    x3 = x.reshape(ROWS // BS, BS, N)
    acc = x3[:, 0, :]
    for r in range(1, BS):
        acc = acc + x3[:, r, :]
    v = acc
    for d in (8, 4, 2, 1):
        v = v + pltpu.roll(v, N - d, 1)
    S = _sel_matrix(N, NB, lambda a, b: a == b * BS)
    s = _dot(v, S)
    scores[pl.ds(pl.multiple_of(i * BS, BS), BS), :] = s

    @pl.when(i == GRID - 1)
    def _():
        sc = scores[...]
        b = lax.bitcast_convert_type(sc, jnp.int32)
        mag = b & jnp.int32(0x7FFFFFFF)
        m = jnp.where(b >= 0, b, jnp.int32(-1) - mag)
        cnt_pos = jnp.sum((m >= 0).astype(jnp.int32))
        cur0 = jnp.where(cnt_pos >= K, jnp.int32(0), jnp.int32(-2147483648))

        def body(t, cur):
            cand = cur + (jnp.int32(1) << (30 - t))
            cnt = jnp.sum((m >= cand).astype(jnp.int32))
            return jnp.where(cnt >= K, cand, cur)

        T = lax.fori_loop(0, 31, body, cur0)
        gt = m > T
        eq = m == T
        need = jnp.float32(K) - jnp.sum(gt.astype(jnp.float32))
        eq_f = eq.astype(jnp.float32)
        rs = jnp.sum(eq_f, axis=1, keepdims=True)
        Lm = _sel_matrix(NB, NB, lambda a, b: a > b)
        Um = _sel_matrix(NB, NB, lambda a, b: a < b)
        row_off = _dot(Lm, rs)
        W = _dot(eq_f, Um)
        prefix = row_off + W
        tie = eq & (prefix < need)
        mask_ref[...] = jnp.where(gt | tie, 1.0, 0.0).astype(jnp.float32)


def _expand_kernel(mask_ref, out_ref):
    i = pl.program_id(0)
    sub = mask_ref[pl.ds(pl.multiple_of(i * BS, BS), BS), :]
    E = _sel_matrix(NB, N, lambda b, j: b == j // BS)
    ex = _dot(sub, E)
    R = _sel_matrix(ROWS, BS, lambda r, c: r // BS == c)
    out_ref[...] = _dot(R, ex)


def kernel(weight, grad):
    mask = pl.pallas_call(
        _scores_mask_kernel,
        grid=(GRID,),
        in_specs=[pl.BlockSpec((ROWS, N), lambda i: (i, 0))],
        out_specs=pl.BlockSpec((NB, NB), lambda i: (0, 0)),
        out_shape=jax.ShapeDtypeStruct((NB, NB), jnp.float32),
        scratch_shapes=[pltpu.VMEM((NB, NB), jnp.float32)],
        compiler_params=pltpu.CompilerParams(
            dimension_semantics=("arbitrary",)),
    )(grad)
    out = pl.pallas_call(
        _expand_kernel,
        grid=(GRID,),
        in_specs=[pl.BlockSpec((NB, NB), lambda i: (0, 0))],
        out_specs=pl.BlockSpec((ROWS, N), lambda i: (i, 0)),
        out_shape=jax.ShapeDtypeStruct((M, N), jnp.float32),
        compiler_params=pltpu.CompilerParams(
            dimension_semantics=("arbitrary",)),
    )(mask)
    return out.astype(weight.dtype)


# speedup vs baseline: 6.9866x; 1.8539x over previous
import jax
import jax.numpy as jnp
from jax import lax
from jax.experimental import pallas as pl
from jax.experimental.pallas import tpu as pltpu

M = N = 4096
BS = 16
NB = M // BS
K = int(int(M * N * 0.05) / (BS * BS))
GRID = 16
ROWS = M // GRID


def _sel_matrix(rows, cols, fn):
    i = lax.broadcasted_iota(jnp.int32, (rows, cols), 0)
    j = lax.broadcasted_iota(jnp.int32, (rows, cols), 1)
    return jnp.where(fn(i, j), 1.0, 0.0).astype(jnp.float32)


def _dot(a, b, precision=lax.Precision.HIGHEST):
    return lax.dot_general(a, b, (((1,), (0,)), ((), ())),
                           precision=precision,
                           preferred_element_type=jnp.float32)


def _scores_mask_kernel(grad_ref, mask_ref, vwide):
    i = pl.program_id(0)
    x = grad_ref[...]
    x3 = x.reshape(ROWS // BS, BS, N)
    acc = x3[:, 0, :]
    for r in range(1, BS):
        acc = acc + x3[:, r, :]
    v = acc
    for d in (8, 4, 2, 1):
        v = v + jnp.concatenate([v[:, d:], v[:, :d]], axis=1)
    vwide[pl.ds(pl.multiple_of(i * BS, BS), BS), :] = v

    @pl.when(i == GRID - 1)
    def _():
        S = _sel_matrix(N, NB, lambda a, b: a == b * BS)
        sc = _dot(vwide[...], S)
        b = lax.bitcast_convert_type(sc, jnp.int32)
        mag = b & jnp.int32(0x7FFFFFFF)
        m = jnp.where(b >= 0, b, jnp.int32(-1) - mag)
        cnt_pos = jnp.sum((m >= 0).astype(jnp.int32))
        cur0 = jnp.where(cnt_pos >= K, jnp.int32(0), jnp.int32(-2147483648))

        def body(t, cur):
            cand = cur + (jnp.int32(1) << (30 - t))
            cnt = jnp.sum((m >= cand).astype(jnp.int32))
            return jnp.where(cnt >= K, cand, cur)

        T = lax.fori_loop(0, 31, body, cur0)
        gt = m > T
        eq = m == T
        need = jnp.float32(K) - jnp.sum(gt.astype(jnp.float32))
        eq_f = eq.astype(jnp.float32)
        rs = jnp.sum(eq_f, axis=1, keepdims=True)
        Lm = _sel_matrix(NB, NB, lambda a, b: a > b)
        Um = _sel_matrix(NB, NB, lambda a, b: a < b)
        row_off = _dot(Lm, rs)
        W = _dot(eq_f, Um)
        prefix = row_off + W
        tie = eq & (prefix < need)
        mask_ref[...] = jnp.where(gt | tie, 1.0, 0.0).astype(jnp.float32)


def _expand_kernel(mask_ref, out_ref):
    i = pl.program_id(0)
    sub = mask_ref[pl.ds(pl.multiple_of(i * BS, BS), BS), :]
    E = _sel_matrix(NB, N, lambda b, j: b == j // BS)
    ex = _dot(sub, E, lax.Precision.DEFAULT)
    R = _sel_matrix(ROWS, BS, lambda r, c: r // BS == c)
    out_ref[...] = _dot(R, ex, lax.Precision.DEFAULT)


def kernel(weight, grad):
    mask = pl.pallas_call(
        _scores_mask_kernel,
        grid=(GRID,),
        in_specs=[pl.BlockSpec((ROWS, N), lambda i: (i, 0))],
        out_specs=pl.BlockSpec((NB, NB), lambda i: (0, 0)),
        out_shape=jax.ShapeDtypeStruct((NB, NB), jnp.float32),
        scratch_shapes=[pltpu.VMEM((NB, N), jnp.float32)],
        compiler_params=pltpu.CompilerParams(
            dimension_semantics=("arbitrary",)),
    )(grad)
    out = pl.pallas_call(
        _expand_kernel,
        grid=(GRID,),
        in_specs=[pl.BlockSpec((NB, NB), lambda i: (0, 0))],
        out_specs=pl.BlockSpec((ROWS, N), lambda i: (i, 0)),
        out_shape=jax.ShapeDtypeStruct((M, N), jnp.float32),
        compiler_params=pltpu.CompilerParams(
            dimension_semantics=("arbitrary",)),
    )(mask)
    return out.astype(weight.dtype)


# speedup vs baseline: 7.8692x; 1.1263x over previous
import jax
import jax.numpy as jnp
from jax import lax
from jax.experimental import pallas as pl
from jax.experimental.pallas import tpu as pltpu

M = N = 4096
BS = 16
NB = M // BS
K = int(int(M * N * 0.05) / (BS * BS))
GRID = 16
ROWS = M // GRID


def _sel_matrix(rows, cols, fn):
    i = lax.broadcasted_iota(jnp.int32, (rows, cols), 0)
    j = lax.broadcasted_iota(jnp.int32, (rows, cols), 1)
    return jnp.where(fn(i, j), 1.0, 0.0).astype(jnp.float32)


def _dot(a, b, precision=lax.Precision.HIGHEST):
    return lax.dot_general(a, b, (((1,), (0,)), ((), ())),
                           precision=precision,
                           preferred_element_type=jnp.float32)


def _rowsum_kernel(grad_ref, out_ref):
    x = grad_ref[...]
    x3 = x.reshape(ROWS // BS, BS, N)
    acc = x3[:, 0, :]
    for r in range(1, BS):
        acc = acc + x3[:, r, :]
    out_ref[...] = acc


def _mask_expand_kernel(planes_ref, out_ref, mask):
    i = pl.program_id(0)

    @pl.when(i == 0)
    def _():
        P = [planes_ref[p] for p in range(BS)]
        P = [P[p] + P[p + 8] for p in range(8)]
        P = [P[p] + P[p + 4] for p in range(4)]
        P = [P[p] + P[p + 2] for p in range(2)]
        sc = P[0] + P[1]

        b = lax.bitcast_convert_type(sc, jnp.int32)
        mag = b & jnp.int32(0x7FFFFFFF)
        m = jnp.where(b >= 0, b, jnp.int32(-1) - mag)
        cnt_pos = jnp.sum((m >= 0).astype(jnp.int32))
        cur0 = jnp.where(cnt_pos >= K, jnp.int32(0), jnp.int32(-2147483648))

        def body(t, cur):
            cand = cur + (jnp.int32(1) << (30 - t))
            cnt = jnp.sum((m >= cand).astype(jnp.int32))
            return jnp.where(cnt >= K, cand, cur)

        T = lax.fori_loop(0, 31, body, cur0)
        gt = m > T
        eq = m == T
        need = jnp.float32(K) - jnp.sum(gt.astype(jnp.float32))
        eq_f = eq.astype(jnp.float32)
        rs = jnp.sum(eq_f, axis=1, keepdims=True)
        Lm = _sel_matrix(NB, NB, lambda a, b2: a > b2)
        Um = _sel_matrix(NB, NB, lambda a, b2: a < b2)
        row_off = _dot(Lm, rs)
        W = _dot(eq_f, Um)
        prefix = row_off + W
        tie = eq & (prefix < need)
        mask[...] = jnp.where(gt | tie, 1.0, 0.0).astype(jnp.float32)

    sub = mask[pl.ds(pl.multiple_of(i * BS, BS), BS), :]
    E = _sel_matrix(NB, N, lambda b2, j: b2 == j // BS)
    ex = _dot(sub, E, lax.Precision.DEFAULT)
    R = _sel_matrix(ROWS, BS, lambda r, c: r // BS == c)
    out_ref[...] = _dot(R, ex, lax.Precision.DEFAULT)


def kernel(weight, grad):
    wide = pl.pallas_call(
        _rowsum_kernel,
        grid=(GRID,),
        in_specs=[pl.BlockSpec((ROWS, N), lambda i: (i, 0))],
        out_specs=pl.BlockSpec((BS, N), lambda i: (i, 0)),
        out_shape=jax.ShapeDtypeStruct((NB, N), jnp.float32),
        compiler_params=pltpu.CompilerParams(
            dimension_semantics=("arbitrary",)),
    )(grad)
    planes = jnp.transpose(wide.reshape(NB, NB, BS), (2, 0, 1))
    out = pl.pallas_call(
        _mask_expand_kernel,
        grid=(GRID,),
        in_specs=[pl.BlockSpec((BS, NB, NB), lambda i: (0, 0, 0))],
        out_specs=pl.BlockSpec((ROWS, N), lambda i: (i, 0)),
        out_shape=jax.ShapeDtypeStruct((M, N), jnp.float32),
        scratch_shapes=[pltpu.VMEM((NB, NB), jnp.float32)],
        compiler_params=pltpu.CompilerParams(
            dimension_semantics=("arbitrary",)),
    )(planes)
    return out.astype(weight.dtype)


# speedup vs baseline: 8.9348x; 1.1354x over previous
import jax
import jax.numpy as jnp
from jax import lax
from jax.experimental import pallas as pl
from jax.experimental.pallas import tpu as pltpu

M = N = 4096
BS = 16
NB = M // BS
K = int(int(M * N * 0.05) / (BS * BS))
GRID = 16
ROWS = M // GRID


def _sel_matrix(rows, cols, fn):
    i = lax.broadcasted_iota(jnp.int32, (rows, cols), 0)
    j = lax.broadcasted_iota(jnp.int32, (rows, cols), 1)
    return jnp.where(fn(i, j), 1.0, 0.0).astype(jnp.float32)


def _dot(a, b, precision=lax.Precision.HIGHEST):
    return lax.dot_general(a, b, (((1,), (0,)), ((), ())),
                           precision=precision,
                           preferred_element_type=jnp.float32)


def _scores_kernel(grad_ref, out_ref):
    x = grad_ref[...]
    x5 = x.reshape(NB, BS, ROWS)
    acc = x5[:, 0, :]
    for r in range(1, BS):
        acc = acc + x5[:, r, :]
    accT = lax.transpose(acc, (1, 0))
    x6 = accT.reshape(BS, BS, NB)
    t = x6[:, 0:8, :] + x6[:, 8:16, :]
    t = t[:, 0:4, :] + t[:, 4:8, :]
    t = t[:, 0:2, :] + t[:, 2:4, :]
    t = t[:, 0:1, :] + t[:, 1:2, :]
    out_ref[...] = t.reshape(BS, NB)


def _mask_expand_kernel(scT_ref, out_ref, mask):
    i = pl.program_id(0)

    @pl.when(i == 0)
    def _():
        sc = scT_ref[...]
        b = lax.bitcast_convert_type(sc, jnp.int32)
        mag = b & jnp.int32(0x7FFFFFFF)
        m = jnp.where(b >= 0, b, jnp.int32(-1) - mag)
        cnt_pos = jnp.sum((m >= 0).astype(jnp.int32))
        cur0 = jnp.where(cnt_pos >= K, jnp.int32(0), jnp.int32(-2147483648))

        def body(t, cur):
            cand = cur + (jnp.int32(1) << (30 - t))
            cnt = jnp.sum((m >= cand).astype(jnp.int32))
            return jnp.where(cnt >= K, cand, cur)

        T = lax.fori_loop(0, 31, body, cur0)
        gt = m > T
        eq = m == T
        need = jnp.float32(K) - jnp.sum(gt.astype(jnp.float32))
        eq_f = eq.astype(jnp.float32)
        s = _dot(jnp.ones((1, NB), jnp.float32), eq_f,
                 lax.Precision.DEFAULT)
        Um = _sel_matrix(NB, NB, lambda a, b2: a < b2)
        w1 = _dot(s, Um, lax.Precision.DEFAULT)
        Lm = _sel_matrix(NB, NB, lambda a, b2: a > b2)
        W2 = _dot(Lm, eq_f, lax.Precision.DEFAULT)
        prefix = w1 + W2
        tie = eq & (prefix < need)
        maskT = jnp.where(gt | tie, 1.0, 0.0).astype(jnp.float32)
        I = _sel_matrix(NB, NB, lambda a, b2: a == b2)
        mask[...] = lax.dot_general(maskT, I, (((0,), (0,)), ((), ())),
                                    precision=lax.Precision.HIGHEST,
                                    preferred_element_type=jnp.float32)

    sub = mask[pl.ds(pl.multiple_of(i * BS, BS), BS), :]
    E = _sel_matrix(NB, N, lambda b2, j: b2 == j // BS)
    ex = _dot(sub, E, lax.Precision.DEFAULT)
    R = _sel_matrix(ROWS, BS, lambda r, c: r // BS == c)
    out_ref[...] = _dot(R, ex, lax.Precision.DEFAULT)


def kernel(weight, grad):
    scT = pl.pallas_call(
        _scores_kernel,
        grid=(GRID,),
        in_specs=[pl.BlockSpec((M, ROWS), lambda i: (0, i))],
        out_specs=pl.BlockSpec((BS, NB), lambda i: (i, 0)),
        out_shape=jax.ShapeDtypeStruct((NB, NB), jnp.float32),
        compiler_params=pltpu.CompilerParams(
            dimension_semantics=("arbitrary",)),
    )(grad)
    out = pl.pallas_call(
        _mask_expand_kernel,
        grid=(GRID,),
        in_specs=[pl.BlockSpec((NB, NB), lambda i: (0, 0))],
        out_specs=pl.BlockSpec((ROWS, N), lambda i: (i, 0)),
        out_shape=jax.ShapeDtypeStruct((M, N), jnp.float32),
        scratch_shapes=[pltpu.VMEM((NB, NB), jnp.float32)],
        compiler_params=pltpu.CompilerParams(
            dimension_semantics=("arbitrary",)),
    )(scT)
    return out.astype(weight.dtype)
